# baseline (speedup 1.0000x reference)
import functools

import jax
import jax.numpy as jnp
from jax.experimental import pallas as pl
from jax.experimental.pallas import tpu as pltpu
from jax.experimental.pallas import tpu_sc as plsc

H = 768
WD = 128
NW = 64
NC = 67
B, T = 8, 512
NA, NP = 2048, 512
KA, KP = 30, 10
NT = 512

F32 = jnp.float32


def _dot_d(a, b):
    return jnp.dot(a, b, preferred_element_type=F32)


def _dot_x(a, b):
    return jnp.dot(a, b, preferred_element_type=F32,
                   precision=jax.lax.Precision.HIGHEST)


def _dot_g(oh, mat):
    hi = mat.astype(jnp.bfloat16).astype(F32)
    r = mat - hi
    mid = r.astype(jnp.bfloat16).astype(F32)
    lolo = r - mid
    return _dot_d(oh, hi) + _dot_d(oh, mid) + _dot_d(oh, lolo)


_NWK = 32
_CH = 64


def _sc_gather(feat_flat, gs_all, ge_all):
    nsp = gs_all.shape[0]
    _SPW = nsp // _NWK
    mesh = plsc.VectorSubcoreMesh(core_axis_name="c", subcore_axis_name="s")
    nch = _SPW // _CH

    @functools.partial(
        pl.kernel, mesh=mesh,
        out_type=[jax.ShapeDtypeStruct((nsp, H // 2), F32),
                  jax.ShapeDtypeStruct((nsp, H // 2), F32)],
        scratch_types=[pltpu.VMEM((_SPW,), jnp.int32),
                       pltpu.VMEM((_SPW,), jnp.int32),
                       pltpu.VMEM((_CH, H // 2), F32),
                       pltpu.VMEM((_CH, H // 2), F32),
                       pltpu.SemaphoreType.DMA,
                       pltpu.SemaphoreType.DMA],
    )
    def k(feat_hbm, gs_hbm, ge_hbm, gs_out, ge_out, gs_v, ge_v, rows0, rows1,
          sem0, sem1):
        wid = jax.lax.axis_index("s") * 2 + jax.lax.axis_index("c")
        base = wid * _SPW
        pltpu.sync_copy(gs_hbm.at[pl.ds(base, _SPW)], gs_v)
        pltpu.sync_copy(ge_hbm.at[pl.ds(base, _SPW)], ge_v)
        bufs = (rows0, rows1)
        sems = (sem0, sem1)
        chunks = []
        for c in range(2 * nch):
            idx_v, out = (gs_v, gs_out) if c < nch else (ge_v, ge_out)
            off = (c % nch) * _CH
            chunks.append((idx_v, out, off))
        pend = None
        for c, (idx_v, out, off) in enumerate(chunks):
            cp = pltpu.async_copy(feat_hbm.at[idx_v.at[pl.ds(off, _CH)]],
                                  bufs[c % 2], sems[c % 2])
            if pend is not None:
                pidx_v, pout, poff, pcp = pend
                pcp.wait()
                pltpu.sync_copy(bufs[(c - 1) % 2],
                                pout.at[pl.ds(base + poff, _CH)])
            pend = (idx_v, out, off, cp)
        pidx_v, pout, poff, pcp = pend
        pcp.wait()
        pltpu.sync_copy(bufs[(2 * nch - 1) % 2],
                        pout.at[pl.ds(base + poff, _CH)])

    return k(feat_flat, gs_all, ge_all)


def _k1_body(feat, att_w, att_b, p1w, p1b, p2w, p2b, pcand, lg_o, psc_o):
    f = feat[0]
    lg = _dot_d(f, att_w[...]) + att_b[0, 0]
    lg_o[0] = lg.T
    fp = _dot_d(f, p1w[0:H]) + _dot_d(f, p1w[H:2 * H]) + p1b[...]
    psp = _dot_d(jnp.maximum(fp, 0.0), p2w[...]) + p2b[0, 0]
    p = pcand[0, :, 0]
    iota = jax.lax.broadcasted_iota(jnp.int32, (NP, T), 1)
    ohp = (iota == p[:, None]).astype(F32)
    psc_o[0] = _dot_g(ohp, psp).T


def _k2_body(feat, gs, ge, wemb, m1ws, m1we, m1ww, m1wa, lg, acand, m1b, m2w,
             m2b, out):
    f = feat[0]
    s = acand[0, :, 0]
    e = acand[0, :, 1]
    w = e - s
    iota = jax.lax.broadcasted_iota(jnp.int32, (NT, T), 1)
    sc_ = s[:, None]
    ec_ = e[:, None]
    inwin = (iota >= sc_) & (iota <= ec_)
    ml = jnp.where(inwin, lg[0], -1e30)
    rowmax = jnp.max(ml, axis=1, keepdims=True)
    ex = jnp.exp(ml - rowmax)
    wmat = ex / jnp.sum(ex, axis=1, keepdims=True)
    attended = _dot_d(wmat, f)
    iw = jax.lax.broadcasted_iota(jnp.int32, (NT, NW), 1)
    ohw = (iw == w[:, None]).astype(F32)
    we1 = _dot_d(wemb[...], m1ww[...])
    acc = (_dot_d(gs[0], m1ws[...]) + _dot_d(ge[0], m1we[...])
           + _dot_g(ohw, we1) + _dot_d(attended, m1wa[...]) + m1b[...])
    h = jnp.maximum(acc, 0.0)
    out[0] = (_dot_d(h, m2w[...]) + m2b[0, 0]).T


def _topk_select(scores, k, n):
    iota = jax.lax.broadcasted_iota(jnp.int32, (B, n), 1).astype(F32)
    cur = scores
    idxs = []
    for _ in range(k):
        m = jnp.max(cur, axis=1, keepdims=True)
        cand = jnp.where(cur == m, iota, float(n))
        idx = jnp.min(cand, axis=1, keepdims=True)
        idxs.append(idx)
        cur = jnp.where(iota == idx, -jnp.inf, cur)
    top = jnp.concatenate(idxs, axis=1)
    rank = jnp.zeros((B, k), F32)
    for j in range(k):
        rank = rank + (top[:, j:j + 1] < top).astype(F32)
    kio = jax.lax.broadcasted_iota(jnp.int32, (B, k), 1).astype(F32)
    srt = jnp.zeros((B, k), F32)
    for i in range(k):
        srt = srt + jnp.where(rank[:, i:i + 1] == kio, top[:, i:i + 1], 0.0)
    return srt.astype(jnp.int32)


def _k3_body(ascores, pscores, ta_o, tp_o):
    ta_o[...] = _topk_select(ascores[...], KA, NA)
    tp_o[...] = _topk_select(pscores[...], KP, NP)


def _k4_body(feat, lg, acand, pcand, ascores, pscores, ta, tp, wemb,
             s1pred, s1arg, s1b, s2w, s2b, out, aspan_o, pspan_o):
    f = feat[0]
    ai = ta[0, 0, :]
    pi = tp[0, 0, :]

    iota_a = jax.lax.broadcasted_iota(jnp.int32, (KA, NA), 1)
    oh_ai = (iota_a == ai[:, None]).astype(F32)
    av = jnp.concatenate([acand[0].astype(F32), ascores[0].T], axis=1)
    ag = _dot_g(oh_ai, av)
    aspan = ag[:, 0:2]
    aspan_o[0] = aspan.astype(jnp.int32)
    asc = ag[:, 2:3]

    iota_p = jax.lax.broadcasted_iota(jnp.int32, (KP, NP), 1)
    oh_pi = (iota_p == pi[:, None]).astype(F32)
    pv = jnp.concatenate([pcand[0].astype(F32), pscores[0].T], axis=1)
    pg = _dot_g(oh_pi, pv)
    pspan = pg[:, 0:2]
    pspan_o[0] = pspan.astype(jnp.int32)
    psc = pg[:, 2:3]

    s_t = aspan[:, 0:1].astype(jnp.int32)
    e_t = aspan[:, 1:2].astype(jnp.int32)
    w_t = e_t - s_t
    iota_t = jax.lax.broadcasted_iota(jnp.int32, (KA, T), 1)
    inwin = (iota_t >= s_t) & (iota_t <= e_t)
    ml = jnp.where(inwin, lg[0], -1e30)
    rowmax = jnp.max(ml, axis=1, keepdims=True)
    ex = jnp.exp(ml - rowmax)
    wmat = ex / jnp.sum(ex, axis=1, keepdims=True)
    att = _dot_d(wmat, f)
    ohs = (iota_t == s_t).astype(F32)
    ohe = (iota_t == e_t).astype(F32)
    fs_t = _dot_g(ohs, f)
    fe_t = _dot_g(ohe, f)
    iw = jax.lax.broadcasted_iota(jnp.int32, (KA, NW), 1)
    ohw = (iw == w_t).astype(F32)
    wd_t = _dot_g(ohw, wemb[...])
    arg_emb = jnp.concatenate([fs_t, fe_t, wd_t, att], axis=1)
    apart = _dot_d(arg_emb, s1arg[...])

    p_t = pspan[:, 0:1].astype(jnp.int32)
    iota_tp = jax.lax.broadcasted_iota(jnp.int32, (KP, T), 1)
    ohp = (iota_tp == p_t).astype(F32)
    fp_t = _dot_g(ohp, f)
    ppart = _dot_d(jnp.concatenate([fp_t, fp_t], axis=1), s1pred[...])

    npair = KP * KA
    ip = jax.lax.broadcasted_iota(jnp.int32, (npair, KP), 0) // KA
    e1 = (ip == jax.lax.broadcasted_iota(jnp.int32, (npair, KP), 1)).astype(F32)
    ia = jax.lax.broadcasted_iota(jnp.int32, (npair, KA), 0) % KA
    e2 = (ia == jax.lax.broadcasted_iota(jnp.int32, (npair, KA), 1)).astype(F32)
    pre = _dot_g(e1, ppart) + _dot_g(e2, apart) + s1b[...]
    h = jnp.maximum(pre, 0.0)
    sc = _dot_d(h, s2w[...]) + s2b[...]
    sc = sc + _dot_g(e2, asc) + _dot_g(e1, psc)
    out[0] = jnp.concatenate([jnp.zeros((npair, 1), F32), sc], axis=1)


def _pipeline(features, arg_candidates, predicate_candidates, width_emb, att_w,
              att_b, m1w, m1b, m2w, m2b, p1w, p1b, p2w, p2b, s1w, s1b, s2w,
              s2b):
    m1ws = m1w[0:H]
    m1we = m1w[H:2 * H]
    m1ww = m1w[2 * H:2 * H + WD]
    m1wa = m1w[2 * H + WD:]
    att_b2 = att_b.reshape(1, 1)
    m1b2 = m1b.reshape(1, H)
    m2b2 = m2b.reshape(1, 1)
    p1b2 = p1b.reshape(1, H)
    p2b2 = p2b.reshape(1, 1)
    s1b2 = s1b.reshape(1, H)
    s2b2 = s2b.reshape(1, NC - 1)
    s1pred = s1w[0:2 * H]
    s1arg = s1w[2 * H:]

    full = lambda shape: pl.BlockSpec(shape, lambda b: (0,) * len(shape))
    fullt = lambda shape: pl.BlockSpec(shape, lambda b, t: (0,) * len(shape))

    boff = (jnp.arange(B, dtype=jnp.int32) * T)[:, None]
    gs_all = (arg_candidates[:, :, 0] + boff).reshape(B * NA)
    ge_all = (arg_candidates[:, :, 1] + boff).reshape(B * NA)
    feat_bf = features.astype(jnp.bfloat16).reshape(B * T, H // 2, 2)
    feat_pk = jax.lax.bitcast_convert_type(feat_bf, F32)
    ga_s, ga_e = _sc_gather(feat_pk, gs_all, ge_all)
    ga_s = jax.lax.bitcast_convert_type(ga_s, jnp.bfloat16).reshape(B, NA, H)
    ga_e = jax.lax.bitcast_convert_type(ga_e, jnp.bfloat16).reshape(B, NA, H)

    lg, psc = pl.pallas_call(
        _k1_body,
        grid=(B,),
        in_specs=[
            pl.BlockSpec((1, T, H), lambda b: (b, 0, 0)),
            full((H, 1)), full((1, 1)),
            full((2 * H, H)), full((1, H)), full((H, 1)), full((1, 1)),
            pl.BlockSpec((1, NP, 2), lambda b: (b, 0, 0)),
        ],
        out_specs=[
            pl.BlockSpec((1, 1, T), lambda b: (b, 0, 0)),
            pl.BlockSpec((1, 1, NP), lambda b: (b, 0, 0)),
        ],
        out_shape=[
            jax.ShapeDtypeStruct((B, 1, T), F32),
            jax.ShapeDtypeStruct((B, 1, NP), F32),
        ],
    )(features, att_w, att_b2, p1w, p1b2, p2w, p2b2, predicate_candidates)

    ascores = pl.pallas_call(
        _k2_body,
        grid=(B, NA // NT),
        in_specs=[
            pl.BlockSpec((1, T, H), lambda b, t: (b, 0, 0)),
            pl.BlockSpec((1, NT, H), lambda b, t: (b, t, 0)),
            pl.BlockSpec((1, NT, H), lambda b, t: (b, t, 0)),
            fullt((NW, WD)), fullt((H, H)), fullt((H, H)),
            fullt((WD, H)), fullt((H, H)),
            pl.BlockSpec((1, 1, T), lambda b, t: (b, 0, 0)),
            pl.BlockSpec((1, NT, 2), lambda b, t: (b, t, 0)),
            fullt((1, H)), fullt((H, 1)), fullt((1, 1)),
        ],
        out_specs=pl.BlockSpec((1, 1, NT), lambda b, t: (b, 0, t)),
        out_shape=jax.ShapeDtypeStruct((B, 1, NA), F32),
    )(features, ga_s, ga_e, width_emb, m1ws, m1we, m1ww, m1wa, lg,
      arg_candidates, m1b2, m2w, m2b2)

    ta, tp = pl.pallas_call(
        _k3_body,
        in_specs=[pl.BlockSpec((B, NA), lambda: (0, 0)),
                  pl.BlockSpec((B, NP), lambda: (0, 0))],
        out_specs=[pl.BlockSpec((B, KA), lambda: (0, 0)),
                   pl.BlockSpec((B, KP), lambda: (0, 0))],
        out_shape=[jax.ShapeDtypeStruct((B, KA), jnp.int32),
                   jax.ShapeDtypeStruct((B, KP), jnp.int32)],
    )(ascores.reshape(B, NA), psc.reshape(B, NP))

    out, aspan, pspan = pl.pallas_call(
        _k4_body,
        grid=(B,),
        in_specs=[
            pl.BlockSpec((1, T, H), lambda b: (b, 0, 0)),
            pl.BlockSpec((1, 1, T), lambda b: (b, 0, 0)),
            pl.BlockSpec((1, NA, 2), lambda b: (b, 0, 0)),
            pl.BlockSpec((1, NP, 2), lambda b: (b, 0, 0)),
            pl.BlockSpec((1, 1, NA), lambda b: (b, 0, 0)),
            pl.BlockSpec((1, 1, NP), lambda b: (b, 0, 0)),
            pl.BlockSpec((1, 1, KA), lambda b: (b, 0, 0)),
            pl.BlockSpec((1, 1, KP), lambda b: (b, 0, 0)),
            full((NW, WD)),
            full((2 * H, H)), full((3 * H + WD, H)),
            full((1, H)), full((H, NC - 1)), full((1, NC - 1)),
        ],
        out_specs=[
            pl.BlockSpec((1, KP * KA, NC), lambda b: (b, 0, 0)),
            pl.BlockSpec((1, KA, 2), lambda b: (b, 0, 0)),
            pl.BlockSpec((1, KP, 2), lambda b: (b, 0, 0)),
        ],
        out_shape=[
            jax.ShapeDtypeStruct((B, KP * KA, NC), F32),
            jax.ShapeDtypeStruct((B, KA, 2), jnp.int32),
            jax.ShapeDtypeStruct((B, KP, 2), jnp.int32),
        ],
    )(features, lg, arg_candidates, predicate_candidates, ascores, psc,
      ta.reshape(B, 1, KA), tp.reshape(B, 1, KP), width_emb,
      s1pred, s1arg, s1b2, s2w, s2b2)

    return (out.reshape(B, KP, KA, NC), pspan, aspan), (ascores, psc, ta, tp)


def kernel(features, arg_candidates, predicate_candidates, width_emb, att_w,
           att_b, m1w, m1b, m2w, m2b, p1w, p1b, p2w, p2b, s1w, s1b, s2w, s2b):
    outs, _ = _pipeline(features, arg_candidates, predicate_candidates,
                        width_emb, att_w, att_b, m1w, m1b, m2w, m2b, p1w, p1b,
                        p2w, p2b, s1w, s1b, s2w, s2b)
    return outs


# speedup vs baseline: 2.2624x; 2.2624x over previous
import functools

import jax
import jax.numpy as jnp
from jax.experimental import pallas as pl
from jax.experimental.pallas import tpu as pltpu
from jax.experimental.pallas import tpu_sc as plsc

H = 768
WD = 128
NW = 64
NC = 67
B, T = 8, 512
NA, NP = 2048, 512
KA, KP = 30, 10
NT = 512

F32 = jnp.float32


def _dot_d(a, b):
    return jnp.dot(a, b, preferred_element_type=F32)


def _dot_x(a, b):
    return jnp.dot(a, b, preferred_element_type=F32,
                   precision=jax.lax.Precision.HIGHEST)


def _dot_g(oh, mat):
    hi = mat.astype(jnp.bfloat16).astype(F32)
    r = mat - hi
    mid = r.astype(jnp.bfloat16).astype(F32)
    lolo = r - mid
    return _dot_d(oh, hi) + _dot_d(oh, mid) + _dot_d(oh, lolo)


_NWK = 32
_CH = 64


def _sc_gather(feat_flat, gs_all, ge_all):
    nsp = gs_all.shape[0]
    _SPW = nsp // _NWK
    mesh = plsc.VectorSubcoreMesh(core_axis_name="c", subcore_axis_name="s")
    nch = _SPW // _CH

    @functools.partial(
        pl.kernel, mesh=mesh,
        out_type=[jax.ShapeDtypeStruct((nsp, H), F32),
                  jax.ShapeDtypeStruct((nsp, H), F32)],
        scratch_types=[pltpu.VMEM((_SPW,), jnp.int32),
                       pltpu.VMEM((_SPW,), jnp.int32),
                       pltpu.VMEM((_CH, H), F32),
                       pltpu.VMEM((_CH, H), F32),
                       pltpu.SemaphoreType.DMA,
                       pltpu.SemaphoreType.DMA],
    )
    def k(feat_hbm, gs_hbm, ge_hbm, gs_out, ge_out, gs_v, ge_v, rows0, rows1,
          sem0, sem1):
        wid = jax.lax.axis_index("s") * 2 + jax.lax.axis_index("c")
        base = wid * _SPW
        pltpu.sync_copy(gs_hbm.at[pl.ds(base, _SPW)], gs_v)
        pltpu.sync_copy(ge_hbm.at[pl.ds(base, _SPW)], ge_v)
        bufs = (rows0, rows1)
        sems = (sem0, sem1)
        chunks = []
        for c in range(2 * nch):
            idx_v, out = (gs_v, gs_out) if c < nch else (ge_v, ge_out)
            off = (c % nch) * _CH
            chunks.append((idx_v, out, off))
        pend = None
        for c, (idx_v, out, off) in enumerate(chunks):
            cp = pltpu.async_copy(feat_hbm.at[idx_v.at[pl.ds(off, _CH)]],
                                  bufs[c % 2], sems[c % 2])
            if pend is not None:
                pidx_v, pout, poff, pcp = pend
                pcp.wait()
                pltpu.sync_copy(bufs[(c - 1) % 2],
                                pout.at[pl.ds(base + poff, _CH)])
            pend = (idx_v, out, off, cp)
        pidx_v, pout, poff, pcp = pend
        pcp.wait()
        pltpu.sync_copy(bufs[(2 * nch - 1) % 2],
                        pout.at[pl.ds(base + poff, _CH)])

    return k(feat_flat, gs_all, ge_all)


def _k2_body(feat, gs, ge, wemb, m1ws, m1we, m1ww, m1wa, att_w, att_b, acand,
             m1b, m2w, m2b, out):
    f = feat[0]
    lgr = (_dot_d(f, att_w[...]) + att_b[0, 0]).T
    s = acand[0, :, 0]
    e = acand[0, :, 1]
    w = e - s
    iota = jax.lax.broadcasted_iota(jnp.int32, (NT, T), 1)
    sc_ = s[:, None]
    ec_ = e[:, None]
    inwin = (iota >= sc_) & (iota <= ec_)
    ml = jnp.where(inwin, lgr, -1e30)
    rowmax = jnp.max(ml, axis=1, keepdims=True)
    ex = jnp.exp(ml - rowmax)
    wmat = ex / jnp.sum(ex, axis=1, keepdims=True)
    attended = _dot_d(wmat, f)
    iw = jax.lax.broadcasted_iota(jnp.int32, (NT, NW), 1)
    ohw = (iw == w[:, None]).astype(F32)
    we1 = _dot_d(wemb[...], m1ww[...])
    acc = (_dot_d(gs[0], m1ws[...]) + _dot_d(ge[0], m1we[...])
           + _dot_g(ohw, we1) + _dot_d(attended, m1wa[...]) + m1b[...])
    h = jnp.maximum(acc, 0.0)
    out[0] = (_dot_d(h, m2w[...]) + m2b[0, 0]).T


def _topk_select(scores, k, n):
    iota = jax.lax.broadcasted_iota(jnp.int32, (B, n), 1).astype(F32)
    cur = scores
    idxs = []
    for _ in range(k):
        m = jnp.max(cur, axis=1, keepdims=True)
        cand = jnp.where(cur == m, iota, float(n))
        idx = jnp.min(cand, axis=1, keepdims=True)
        idxs.append(idx)
        cur = jnp.where(iota == idx, -jnp.inf, cur)
    top = jnp.concatenate(idxs, axis=1)
    rank = jnp.zeros((B, k), F32)
    for j in range(k):
        rank = rank + (top[:, j:j + 1] < top).astype(F32)
    kio = jax.lax.broadcasted_iota(jnp.int32, (B, k), 1).astype(F32)
    srt = jnp.zeros((B, k), F32)
    for i in range(k):
        srt = srt + jnp.where(rank[:, i:i + 1] == kio, top[:, i:i + 1], 0.0)
    return srt.astype(jnp.int32)


def _k3_body(feat, p1w, p1b, p2w, p2b, pcand, ascores, ta_o, tp_o, psc_o):
    f = feat[...]
    fp = _dot_d(f, p1w[0:H]) + _dot_d(f, p1w[H:2 * H]) + p1b[...]
    psp = _dot_d(jnp.maximum(fp, 0.0), p2w[...]) + p2b[0, 0]
    rows = []
    for b in range(B):
        p = pcand[b, :, 0]
        iota = jax.lax.broadcasted_iota(jnp.int32, (NP, T), 1)
        ohp = (iota == p[:, None]).astype(F32)
        rows.append(_dot_g(ohp, psp[b * T:(b + 1) * T]).T)
    pm = jnp.concatenate(rows, axis=0)
    psc_o[...] = pm
    ta_o[...] = _topk_select(ascores[...], KA, NA)
    tp_o[...] = _topk_select(pm, KP, NP)


def _k4_body(feat, att_w, att_b, acand, pcand, ascores, pscores, ta, tp, wemb,
             s1pred, s1arg, s1b, s2w, s2b, out, aspan_o, pspan_o):
    f = feat[0]
    lgr = (_dot_d(f, att_w[...]) + att_b[0, 0]).T
    ai = ta[0, 0, :]
    pi = tp[0, 0, :]

    iota_a = jax.lax.broadcasted_iota(jnp.int32, (KA, NA), 1)
    oh_ai = (iota_a == ai[:, None]).astype(F32)
    av = jnp.concatenate([acand[0].astype(F32), ascores[0].T], axis=1)
    ag = _dot_g(oh_ai, av)
    aspan = ag[:, 0:2]
    aspan_o[0] = aspan.astype(jnp.int32)
    asc = ag[:, 2:3]

    iota_p = jax.lax.broadcasted_iota(jnp.int32, (KP, NP), 1)
    oh_pi = (iota_p == pi[:, None]).astype(F32)
    pv = jnp.concatenate([pcand[0].astype(F32), pscores[0].T], axis=1)
    pg = _dot_g(oh_pi, pv)
    pspan = pg[:, 0:2]
    pspan_o[0] = pspan.astype(jnp.int32)
    psc = pg[:, 2:3]

    s_t = aspan[:, 0:1].astype(jnp.int32)
    e_t = aspan[:, 1:2].astype(jnp.int32)
    w_t = e_t - s_t
    iota_t = jax.lax.broadcasted_iota(jnp.int32, (KA, T), 1)
    inwin = (iota_t >= s_t) & (iota_t <= e_t)
    ml = jnp.where(inwin, lgr, -1e30)
    rowmax = jnp.max(ml, axis=1, keepdims=True)
    ex = jnp.exp(ml - rowmax)
    wmat = ex / jnp.sum(ex, axis=1, keepdims=True)
    att = _dot_d(wmat, f)
    ohs = (iota_t == s_t).astype(F32)
    ohe = (iota_t == e_t).astype(F32)
    fs_t = _dot_g(ohs, f)
    fe_t = _dot_g(ohe, f)
    iw = jax.lax.broadcasted_iota(jnp.int32, (KA, NW), 1)
    ohw = (iw == w_t).astype(F32)
    wd_t = _dot_g(ohw, wemb[...])
    arg_emb = jnp.concatenate([fs_t, fe_t, wd_t, att], axis=1)
    apart = _dot_d(arg_emb, s1arg[...])

    p_t = pspan[:, 0:1].astype(jnp.int32)
    iota_tp = jax.lax.broadcasted_iota(jnp.int32, (KP, T), 1)
    ohp = (iota_tp == p_t).astype(F32)
    fp_t = _dot_g(ohp, f)
    ppart = _dot_d(jnp.concatenate([fp_t, fp_t], axis=1), s1pred[...])

    npair = KP * KA
    ip = jax.lax.broadcasted_iota(jnp.int32, (npair, KP), 0) // KA
    e1 = (ip == jax.lax.broadcasted_iota(jnp.int32, (npair, KP), 1)).astype(F32)
    ia = jax.lax.broadcasted_iota(jnp.int32, (npair, KA), 0) % KA
    e2 = (ia == jax.lax.broadcasted_iota(jnp.int32, (npair, KA), 1)).astype(F32)
    pre = _dot_g(e1, ppart) + _dot_g(e2, apart) + s1b[...]
    h = jnp.maximum(pre, 0.0)
    sc = _dot_d(h, s2w[...]) + s2b[...]
    sc = sc + _dot_g(e2, asc) + _dot_g(e1, psc)
    out[0] = jnp.concatenate([jnp.zeros((npair, 1), F32), sc], axis=1)


def _pipeline(features, arg_candidates, predicate_candidates, width_emb, att_w,
              att_b, m1w, m1b, m2w, m2b, p1w, p1b, p2w, p2b, s1w, s1b, s2w,
              s2b):
    m1ws = m1w[0:H]
    m1we = m1w[H:2 * H]
    m1ww = m1w[2 * H:2 * H + WD]
    m1wa = m1w[2 * H + WD:]
    att_b2 = att_b.reshape(1, 1)
    m1b2 = m1b.reshape(1, H)
    m2b2 = m2b.reshape(1, 1)
    p1b2 = p1b.reshape(1, H)
    p2b2 = p2b.reshape(1, 1)
    s1b2 = s1b.reshape(1, H)
    s2b2 = s2b.reshape(1, NC - 1)
    s1pred = s1w[0:2 * H]
    s1arg = s1w[2 * H:]

    full = lambda shape: pl.BlockSpec(shape, lambda b: (0,) * len(shape))
    fullt = lambda shape: pl.BlockSpec(shape, lambda b, t: (0,) * len(shape))

    boff = (jnp.arange(B, dtype=jnp.int32) * T)[:, None]
    gs_all = (arg_candidates[:, :, 0] + boff).reshape(B * NA)
    ge_all = (arg_candidates[:, :, 1] + boff).reshape(B * NA)
    feat_flat = features.reshape(B * T, H)
    ga_s, ga_e = _sc_gather(feat_flat, gs_all, ge_all)
    ga_s = ga_s.reshape(B, NA, H)
    ga_e = ga_e.reshape(B, NA, H)

    ascores = pl.pallas_call(
        _k2_body,
        grid=(B, NA // NT),
        in_specs=[
            pl.BlockSpec((1, T, H), lambda b, t: (b, 0, 0)),
            pl.BlockSpec((1, NT, H), lambda b, t: (b, t, 0)),
            pl.BlockSpec((1, NT, H), lambda b, t: (b, t, 0)),
            fullt((NW, WD)), fullt((H, H)), fullt((H, H)),
            fullt((WD, H)), fullt((H, H)),
            fullt((H, 1)), fullt((1, 1)),
            pl.BlockSpec((1, NT, 2), lambda b, t: (b, t, 0)),
            fullt((1, H)), fullt((H, 1)), fullt((1, 1)),
        ],
        out_specs=pl.BlockSpec((1, 1, NT), lambda b, t: (b, 0, t)),
        out_shape=jax.ShapeDtypeStruct((B, 1, NA), F32),
    )(features, ga_s, ga_e, width_emb, m1ws, m1we, m1ww, m1wa, att_w, att_b2,
      arg_candidates, m1b2, m2w, m2b2)

    nullmap = lambda shape: pl.BlockSpec(shape, lambda: (0,) * len(shape))
    ta, tp, psc = pl.pallas_call(
        _k3_body,
        in_specs=[nullmap((B * T, H)),
                  nullmap((2 * H, H)), nullmap((1, H)), nullmap((H, 1)),
                  nullmap((1, 1)),
                  nullmap((B, NP, 2)),
                  nullmap((B, NA))],
        out_specs=[nullmap((B, KA)), nullmap((B, KP)), nullmap((B, NP))],
        out_shape=[jax.ShapeDtypeStruct((B, KA), jnp.int32),
                   jax.ShapeDtypeStruct((B, KP), jnp.int32),
                   jax.ShapeDtypeStruct((B, NP), F32)],
    )(feat_flat, p1w, p1b2, p2w, p2b2, predicate_candidates,
      ascores.reshape(B, NA))
    psc = psc.reshape(B, 1, NP)

    out, aspan, pspan = pl.pallas_call(
        _k4_body,
        grid=(B,),
        in_specs=[
            pl.BlockSpec((1, T, H), lambda b: (b, 0, 0)),
            full((H, 1)), full((1, 1)),
            pl.BlockSpec((1, NA, 2), lambda b: (b, 0, 0)),
            pl.BlockSpec((1, NP, 2), lambda b: (b, 0, 0)),
            pl.BlockSpec((1, 1, NA), lambda b: (b, 0, 0)),
            pl.BlockSpec((1, 1, NP), lambda b: (b, 0, 0)),
            pl.BlockSpec((1, 1, KA), lambda b: (b, 0, 0)),
            pl.BlockSpec((1, 1, KP), lambda b: (b, 0, 0)),
            full((NW, WD)),
            full((2 * H, H)), full((3 * H + WD, H)),
            full((1, H)), full((H, NC - 1)), full((1, NC - 1)),
        ],
        out_specs=[
            pl.BlockSpec((1, KP * KA, NC), lambda b: (b, 0, 0)),
            pl.BlockSpec((1, KA, 2), lambda b: (b, 0, 0)),
            pl.BlockSpec((1, KP, 2), lambda b: (b, 0, 0)),
        ],
        out_shape=[
            jax.ShapeDtypeStruct((B, KP * KA, NC), F32),
            jax.ShapeDtypeStruct((B, KA, 2), jnp.int32),
            jax.ShapeDtypeStruct((B, KP, 2), jnp.int32),
        ],
    )(features, att_w, att_b2, arg_candidates, predicate_candidates, ascores,
      psc, ta.reshape(B, 1, KA), tp.reshape(B, 1, KP), width_emb,
      s1pred, s1arg, s1b2, s2w, s2b2)

    return (out.reshape(B, KP, KA, NC), pspan, aspan), (ascores, psc, ta, tp)


def kernel(features, arg_candidates, predicate_candidates, width_emb, att_w,
           att_b, m1w, m1b, m2w, m2b, p1w, p1b, p2w, p2b, s1w, s1b, s2w, s2b):
    outs, _ = _pipeline(features, arg_candidates, predicate_candidates,
                        width_emb, att_w, att_b, m1w, m1b, m2w, m2b, p1w, p1b,
                        p2w, p2b, s1w, s1b, s2w, s2b)
    return outs


# speedup vs baseline: 2.4053x; 1.0631x over previous
import functools

import jax
import jax.numpy as jnp
from jax.experimental import pallas as pl
from jax.experimental.pallas import tpu as pltpu
from jax.experimental.pallas import tpu_sc as plsc

H = 768
WD = 128
NW = 64
NC = 67
B, T = 8, 512
NA, NP = 2048, 512
KA, KP = 30, 10
NT = 512

F32 = jnp.float32


def _dot_d(a, b):
    return jnp.dot(a, b, preferred_element_type=F32)


def _dot_x(a, b):
    return jnp.dot(a, b, preferred_element_type=F32,
                   precision=jax.lax.Precision.HIGHEST)


def _dot_g(oh, mat):
    hi = mat.astype(jnp.bfloat16).astype(F32)
    r = mat - hi
    mid = r.astype(jnp.bfloat16).astype(F32)
    lolo = r - mid
    return _dot_d(oh, hi) + _dot_d(oh, mid) + _dot_d(oh, lolo)


_NWK = 32
_CH = 64


def _sc_gather(feat_flat, gs_all, ge_all):
    nsp = gs_all.shape[0]
    _SPW = nsp // _NWK
    mesh = plsc.VectorSubcoreMesh(core_axis_name="c", subcore_axis_name="s")
    nch = _SPW // _CH

    @functools.partial(
        pl.kernel, mesh=mesh,
        out_type=[jax.ShapeDtypeStruct((nsp, H), F32),
                  jax.ShapeDtypeStruct((nsp, H), F32)],
        scratch_types=[pltpu.VMEM((_SPW,), jnp.int32),
                       pltpu.VMEM((_SPW,), jnp.int32),
                       pltpu.VMEM((_CH, H), F32),
                       pltpu.VMEM((_CH, H), F32),
                       pltpu.SemaphoreType.DMA,
                       pltpu.SemaphoreType.DMA],
    )
    def k(feat_hbm, gs_hbm, ge_hbm, gs_out, ge_out, gs_v, ge_v, rows0, rows1,
          sem0, sem1):
        wid = jax.lax.axis_index("s") * 2 + jax.lax.axis_index("c")
        base = wid * _SPW
        pltpu.sync_copy(gs_hbm.at[pl.ds(base, _SPW)], gs_v)
        pltpu.sync_copy(ge_hbm.at[pl.ds(base, _SPW)], ge_v)
        bufs = (rows0, rows1)
        sems = (sem0, sem1)
        chunks = []
        for c in range(2 * nch):
            idx_v, out = (gs_v, gs_out) if c < nch else (ge_v, ge_out)
            off = (c % nch) * _CH
            chunks.append((idx_v, out, off))
        pend = None
        for c, (idx_v, out, off) in enumerate(chunks):
            cp = pltpu.async_copy(feat_hbm.at[idx_v.at[pl.ds(off, _CH)]],
                                  bufs[c % 2], sems[c % 2])
            if pend is not None:
                pidx_v, pout, poff, pcp = pend
                pcp.wait()
                pltpu.sync_copy(bufs[(c - 1) % 2],
                                pout.at[pl.ds(base + poff, _CH)])
            pend = (idx_v, out, off, cp)
        pidx_v, pout, poff, pcp = pend
        pcp.wait()
        pltpu.sync_copy(bufs[(2 * nch - 1) % 2],
                        pout.at[pl.ds(base + poff, _CH)])

    return k(feat_flat, gs_all, ge_all)


def _k1_body(feat, att_w, att_b, p1w, p1b, p2w, p2b, pcand, lg_o, psc_o):
    f = feat[0]
    lg = _dot_d(f, att_w[...]) + att_b[0, 0]
    lg_o[0] = lg.T
    fp = _dot_d(f, p1w[0:H]) + _dot_d(f, p1w[H:2 * H]) + p1b[...]
    psp = _dot_d(jnp.maximum(fp, 0.0), p2w[...]) + p2b[0, 0]
    p = pcand[0, :, 0]
    iota = jax.lax.broadcasted_iota(jnp.int32, (NP, T), 1)
    ohp = (iota == p[:, None]).astype(F32)
    psc_o[0] = _dot_g(ohp, psp).T


def _k2_body(feat, gs, ge, wemb, m1ws, m1we, m1ww, m1wa, lg, acand,
             m1b, m2w, m2b, out):
    f = feat[0]
    lgr = lg[0]
    s = acand[0, :, 0]
    e = acand[0, :, 1]
    w = e - s
    iota = jax.lax.broadcasted_iota(jnp.int32, (NT, T), 1)
    sc_ = s[:, None]
    ec_ = e[:, None]
    inwin = (iota >= sc_) & (iota <= ec_)
    ml = jnp.where(inwin, lgr, -1e30)
    rowmax = jnp.max(ml, axis=1, keepdims=True)
    ex = jnp.exp(ml - rowmax)
    wmat = ex / jnp.sum(ex, axis=1, keepdims=True)
    attended = _dot_d(wmat, f)
    iw = jax.lax.broadcasted_iota(jnp.int32, (NT, NW), 1)
    ohw = (iw == w[:, None]).astype(F32)
    we1 = _dot_d(wemb[...], m1ww[...])
    acc = (_dot_d(gs[0], m1ws[...]) + _dot_d(ge[0], m1we[...])
           + _dot_g(ohw, we1) + _dot_d(attended, m1wa[...]) + m1b[...])
    h = jnp.maximum(acc, 0.0)
    out[0] = (_dot_d(h, m2w[...]) + m2b[0, 0]).T


def _topk_select(scores, k, n):
    iota = jax.lax.broadcasted_iota(jnp.int32, (B, n), 1).astype(F32)
    cur = scores
    idxs = []
    for _ in range(k):
        m = jnp.max(cur, axis=1, keepdims=True)
        cand = jnp.where(cur == m, iota, float(n))
        idx = jnp.min(cand, axis=1, keepdims=True)
        idxs.append(idx)
        cur = jnp.where(iota == idx, -jnp.inf, cur)
    top = jnp.concatenate(idxs, axis=1)
    rank = jnp.zeros((B, k), F32)
    for j in range(k):
        rank = rank + (top[:, j:j + 1] < top).astype(F32)
    kio = jax.lax.broadcasted_iota(jnp.int32, (B, k), 1).astype(F32)
    srt = jnp.zeros((B, k), F32)
    for i in range(k):
        srt = srt + jnp.where(rank[:, i:i + 1] == kio, top[:, i:i + 1], 0.0)
    return srt.astype(jnp.int32)


def _k3_body(ascores, pscores, ta_o, tp_o):
    ta_o[...] = _topk_select(ascores[...], KA, NA)
    tp_o[...] = _topk_select(pscores[...], KP, NP)


def _k4_body(feat, lg, acand, pcand, ascores, pscores, ta, tp, wemb,
             s1pred, s1arg, s1b, s2w, s2b, out, aspan_o, pspan_o):
    f = feat[0]
    lgr = lg[0]
    ai = ta[0, 0, :]
    pi = tp[0, 0, :]

    iota_a = jax.lax.broadcasted_iota(jnp.int32, (KA, NA), 1)
    oh_ai = (iota_a == ai[:, None]).astype(F32)
    av = jnp.concatenate([acand[0].astype(F32), ascores[0].T], axis=1)
    ag = _dot_g(oh_ai, av)
    aspan = ag[:, 0:2]
    aspan_o[0] = aspan.astype(jnp.int32)
    asc = ag[:, 2:3]

    iota_p = jax.lax.broadcasted_iota(jnp.int32, (KP, NP), 1)
    oh_pi = (iota_p == pi[:, None]).astype(F32)
    pv = jnp.concatenate([pcand[0].astype(F32), pscores[0].T], axis=1)
    pg = _dot_g(oh_pi, pv)
    pspan = pg[:, 0:2]
    pspan_o[0] = pspan.astype(jnp.int32)
    psc = pg[:, 2:3]

    s_t = aspan[:, 0:1].astype(jnp.int32)
    e_t = aspan[:, 1:2].astype(jnp.int32)
    w_t = e_t - s_t
    iota_t = jax.lax.broadcasted_iota(jnp.int32, (KA, T), 1)
    inwin = (iota_t >= s_t) & (iota_t <= e_t)
    ml = jnp.where(inwin, lgr, -1e30)
    rowmax = jnp.max(ml, axis=1, keepdims=True)
    ex = jnp.exp(ml - rowmax)
    wmat = ex / jnp.sum(ex, axis=1, keepdims=True)
    att = _dot_d(wmat, f)
    ohs = (iota_t == s_t).astype(F32)
    ohe = (iota_t == e_t).astype(F32)
    fs_t = _dot_g(ohs, f)
    fe_t = _dot_g(ohe, f)
    iw = jax.lax.broadcasted_iota(jnp.int32, (KA, NW), 1)
    ohw = (iw == w_t).astype(F32)
    wd_t = _dot_g(ohw, wemb[...])
    arg_emb = jnp.concatenate([fs_t, fe_t, wd_t, att], axis=1)
    apart = _dot_d(arg_emb, s1arg[...])

    p_t = pspan[:, 0:1].astype(jnp.int32)
    iota_tp = jax.lax.broadcasted_iota(jnp.int32, (KP, T), 1)
    ohp = (iota_tp == p_t).astype(F32)
    fp_t = _dot_g(ohp, f)
    ppart = _dot_d(jnp.concatenate([fp_t, fp_t], axis=1), s1pred[...])

    npair = KP * KA
    ip = jax.lax.broadcasted_iota(jnp.int32, (npair, KP), 0) // KA
    e1 = (ip == jax.lax.broadcasted_iota(jnp.int32, (npair, KP), 1)).astype(F32)
    ia = jax.lax.broadcasted_iota(jnp.int32, (npair, KA), 0) % KA
    e2 = (ia == jax.lax.broadcasted_iota(jnp.int32, (npair, KA), 1)).astype(F32)
    pre = _dot_g(e1, ppart) + _dot_g(e2, apart) + s1b[...]
    h = jnp.maximum(pre, 0.0)
    sc = _dot_d(h, s2w[...]) + s2b[...]
    sc = sc + _dot_g(e2, asc) + _dot_g(e1, psc)
    out[0] = jnp.concatenate([jnp.zeros((npair, 1), F32), sc], axis=1)


def _pipeline(features, arg_candidates, predicate_candidates, width_emb, att_w,
              att_b, m1w, m1b, m2w, m2b, p1w, p1b, p2w, p2b, s1w, s1b, s2w,
              s2b):
    m1ws = m1w[0:H]
    m1we = m1w[H:2 * H]
    m1ww = m1w[2 * H:2 * H + WD]
    m1wa = m1w[2 * H + WD:]
    att_b2 = att_b.reshape(1, 1)
    m1b2 = m1b.reshape(1, H)
    m2b2 = m2b.reshape(1, 1)
    p1b2 = p1b.reshape(1, H)
    p2b2 = p2b.reshape(1, 1)
    s1b2 = s1b.reshape(1, H)
    s2b2 = s2b.reshape(1, NC - 1)
    s1pred = s1w[0:2 * H]
    s1arg = s1w[2 * H:]

    full = lambda shape: pl.BlockSpec(shape, lambda b: (0,) * len(shape))
    fullt = lambda shape: pl.BlockSpec(shape, lambda b, t: (0,) * len(shape))

    boff = (jnp.arange(B, dtype=jnp.int32) * T)[:, None]
    gs_all = (arg_candidates[:, :, 0] + boff).reshape(B * NA)
    ge_all = (arg_candidates[:, :, 1] + boff).reshape(B * NA)
    feat_flat = features.reshape(B * T, H)
    ga_s, ga_e = _sc_gather(feat_flat, gs_all, ge_all)
    ga_s = ga_s.reshape(B, NA, H)
    ga_e = ga_e.reshape(B, NA, H)

    lg, psc = pl.pallas_call(
        _k1_body,
        grid=(B,),
        in_specs=[
            pl.BlockSpec((1, T, H), lambda b: (b, 0, 0)),
            full((H, 1)), full((1, 1)),
            full((2 * H, H)), full((1, H)), full((H, 1)), full((1, 1)),
            pl.BlockSpec((1, NP, 2), lambda b: (b, 0, 0)),
        ],
        out_specs=[
            pl.BlockSpec((1, 1, T), lambda b: (b, 0, 0)),
            pl.BlockSpec((1, 1, NP), lambda b: (b, 0, 0)),
        ],
        out_shape=[
            jax.ShapeDtypeStruct((B, 1, T), F32),
            jax.ShapeDtypeStruct((B, 1, NP), F32),
        ],
    )(features, att_w, att_b2, p1w, p1b2, p2w, p2b2, predicate_candidates)

    ascores = pl.pallas_call(
        _k2_body,
        grid=(B, NA // NT),
        in_specs=[
            pl.BlockSpec((1, T, H), lambda b, t: (b, 0, 0)),
            pl.BlockSpec((1, NT, H), lambda b, t: (b, t, 0)),
            pl.BlockSpec((1, NT, H), lambda b, t: (b, t, 0)),
            fullt((NW, WD)), fullt((H, H)), fullt((H, H)),
            fullt((WD, H)), fullt((H, H)),
            pl.BlockSpec((1, 1, T), lambda b, t: (b, 0, 0)),
            pl.BlockSpec((1, NT, 2), lambda b, t: (b, t, 0)),
            fullt((1, H)), fullt((H, 1)), fullt((1, 1)),
        ],
        out_specs=pl.BlockSpec((1, 1, NT), lambda b, t: (b, 0, t)),
        out_shape=jax.ShapeDtypeStruct((B, 1, NA), F32),
    )(features, ga_s, ga_e, width_emb, m1ws, m1we, m1ww, m1wa, lg,
      arg_candidates, m1b2, m2w, m2b2)

    ta, tp = pl.pallas_call(
        _k3_body,
        in_specs=[pl.BlockSpec((B, NA), lambda: (0, 0)),
                  pl.BlockSpec((B, NP), lambda: (0, 0))],
        out_specs=[pl.BlockSpec((B, KA), lambda: (0, 0)),
                   pl.BlockSpec((B, KP), lambda: (0, 0))],
        out_shape=[jax.ShapeDtypeStruct((B, KA), jnp.int32),
                   jax.ShapeDtypeStruct((B, KP), jnp.int32)],
    )(ascores.reshape(B, NA), psc.reshape(B, NP))

    out, aspan, pspan = pl.pallas_call(
        _k4_body,
        grid=(B,),
        in_specs=[
            pl.BlockSpec((1, T, H), lambda b: (b, 0, 0)),
            pl.BlockSpec((1, 1, T), lambda b: (b, 0, 0)),
            pl.BlockSpec((1, NA, 2), lambda b: (b, 0, 0)),
            pl.BlockSpec((1, NP, 2), lambda b: (b, 0, 0)),
            pl.BlockSpec((1, 1, NA), lambda b: (b, 0, 0)),
            pl.BlockSpec((1, 1, NP), lambda b: (b, 0, 0)),
            pl.BlockSpec((1, 1, KA), lambda b: (b, 0, 0)),
            pl.BlockSpec((1, 1, KP), lambda b: (b, 0, 0)),
            full((NW, WD)),
            full((2 * H, H)), full((3 * H + WD, H)),
            full((1, H)), full((H, NC - 1)), full((1, NC - 1)),
        ],
        out_specs=[
            pl.BlockSpec((1, KP * KA, NC), lambda b: (b, 0, 0)),
            pl.BlockSpec((1, KA, 2), lambda b: (b, 0, 0)),
            pl.BlockSpec((1, KP, 2), lambda b: (b, 0, 0)),
        ],
        out_shape=[
            jax.ShapeDtypeStruct((B, KP * KA, NC), F32),
            jax.ShapeDtypeStruct((B, KA, 2), jnp.int32),
            jax.ShapeDtypeStruct((B, KP, 2), jnp.int32),
        ],
    )(features, lg, arg_candidates, predicate_candidates, ascores,
      psc, ta.reshape(B, 1, KA), tp.reshape(B, 1, KP), width_emb,
      s1pred, s1arg, s1b2, s2w, s2b2)

    return (out.reshape(B, KP, KA, NC), pspan, aspan), (ascores, psc, ta, tp)


def kernel(features, arg_candidates, predicate_candidates, width_emb, att_w,
           att_b, m1w, m1b, m2w, m2b, p1w, p1b, p2w, p2b, s1w, s1b, s2w, s2b):
    outs, _ = _pipeline(features, arg_candidates, predicate_candidates,
                        width_emb, att_w, att_b, m1w, m1b, m2w, m2b, p1w, p1b,
                        p2w, p2b, s1w, s1b, s2w, s2b)
    return outs


# speedup vs baseline: 2.4784x; 1.0304x over previous
import functools

import jax
import jax.numpy as jnp
from jax.experimental import pallas as pl
from jax.experimental.pallas import tpu as pltpu
from jax.experimental.pallas import tpu_sc as plsc

H = 768
WD = 128
NW = 64
NC = 67
B, T = 8, 512
NA, NP = 2048, 512
KA, KP = 30, 10
NT = 1024

F32 = jnp.float32


def _dot_d(a, b):
    return jnp.dot(a, b, preferred_element_type=F32)


def _dot_g(oh, mat):
    hi = mat.astype(jnp.bfloat16).astype(F32)
    r = mat - hi
    mid = r.astype(jnp.bfloat16).astype(F32)
    lolo = r - mid
    return _dot_d(oh, hi) + _dot_d(oh, mid) + _dot_d(oh, lolo)


_NWK = 32
_CH = 64


def _sc_gather(feat_flat, gs_all, ge_all):
    nsp = gs_all.shape[0]
    _SPW = nsp // _NWK
    mesh = plsc.VectorSubcoreMesh(core_axis_name="c", subcore_axis_name="s")
    nch = _SPW // _CH

    @functools.partial(
        pl.kernel, mesh=mesh,
        out_type=[jax.ShapeDtypeStruct((nsp, H), F32),
                  jax.ShapeDtypeStruct((nsp, H), F32)],
        scratch_types=[pltpu.VMEM((_SPW,), jnp.int32),
                       pltpu.VMEM((_SPW,), jnp.int32),
                       pltpu.VMEM((_CH, H), F32),
                       pltpu.VMEM((_CH, H), F32),
                       pltpu.SemaphoreType.DMA,
                       pltpu.SemaphoreType.DMA],
    )
    def k(feat_hbm, gs_hbm, ge_hbm, gs_out, ge_out, gs_v, ge_v, rows0, rows1,
          sem0, sem1):
        wid = jax.lax.axis_index("s") * 2 + jax.lax.axis_index("c")
        base = wid * _SPW
        pltpu.sync_copy(gs_hbm.at[pl.ds(base, _SPW)], gs_v)
        pltpu.sync_copy(ge_hbm.at[pl.ds(base, _SPW)], ge_v)
        bufs = (rows0, rows1)
        sems = (sem0, sem1)
        chunks = []
        for c in range(2 * nch):
            idx_v, out = (gs_v, gs_out) if c < nch else (ge_v, ge_out)
            off = (c % nch) * _CH
            chunks.append((idx_v, out, off))
        pend = None
        for c, (idx_v, out, off) in enumerate(chunks):
            cp = pltpu.async_copy(feat_hbm.at[idx_v.at[pl.ds(off, _CH)]],
                                  bufs[c % 2], sems[c % 2])
            if pend is not None:
                pidx_v, pout, poff, pcp = pend
                pcp.wait()
                pltpu.sync_copy(bufs[(c - 1) % 2],
                                pout.at[pl.ds(base + poff, _CH)])
            pend = (idx_v, out, off, cp)
        pidx_v, pout, poff, pcp = pend
        pcp.wait()
        pltpu.sync_copy(bufs[(2 * nch - 1) % 2],
                        pout.at[pl.ds(base + poff, _CH)])

    return k(feat_flat, gs_all, ge_all)


def _k1_body(feat, att_w, att_b, p1w, p1b, p2w, p2b, pcand, lg_o, psc_o):
    f = feat[0]
    lg = _dot_d(f, att_w[...]) + att_b[0, 0]
    lg_o[0] = lg.T
    fp = _dot_d(f, p1w[0:H]) + _dot_d(f, p1w[H:2 * H]) + p1b[...]
    psp = _dot_d(jnp.maximum(fp, 0.0), p2w[...]) + p2b[0, 0]
    p = pcand[0, :, 0]
    iota = jax.lax.broadcasted_iota(jnp.int32, (NP, T), 1)
    ohp = (iota == p[:, None]).astype(F32)
    psc_o[0] = _dot_g(ohp, psp).T


def _k2_body(feat, gs, ge, wemb, m1ws, m1we, m1ww, m1wa, lg, acand,
             m1b, m2w, m2b, out):
    f = feat[0]
    lgr = lg[0]
    s = acand[0, :, 0]
    e = acand[0, :, 1]
    w = e - s
    iota = jax.lax.broadcasted_iota(jnp.int32, (NT, T), 1)
    sc_ = s[:, None]
    ec_ = e[:, None]
    inwin = (iota >= sc_) & (iota <= ec_)
    ml = jnp.where(inwin, lgr, -1e30)
    rowmax = jnp.max(ml, axis=1, keepdims=True)
    ex = jnp.exp(ml - rowmax)
    wmat = ex / jnp.sum(ex, axis=1, keepdims=True)
    attended = _dot_d(wmat, f)
    iw = jax.lax.broadcasted_iota(jnp.int32, (NT, NW), 1)
    ohw = (iw == w[:, None]).astype(F32)
    we1 = _dot_d(wemb[...], m1ww[...])
    acc = (_dot_d(gs[0], m1ws[...]) + _dot_d(ge[0], m1we[...])
           + _dot_g(ohw, we1) + _dot_d(attended, m1wa[...]) + m1b[...])
    h = jnp.maximum(acc, 0.0)
    out[0] = (_dot_d(h, m2w[...]) + m2b[0, 0]).T


def _topk_select(scores, k, n):
    iota = jax.lax.broadcasted_iota(jnp.int32, (B, n), 1).astype(F32)
    cur = scores
    idxs = []
    for _ in range(k):
        m = jnp.max(cur, axis=1, keepdims=True)
        cand = jnp.where(cur == m, iota, float(n))
        idx = jnp.min(cand, axis=1, keepdims=True)
        idxs.append(idx)
        cur = jnp.where(iota == idx, -jnp.inf, cur)
    top = jnp.concatenate(idxs, axis=1)
    rank = jnp.zeros((B, k), F32)
    for j in range(k):
        rank = rank + (top[:, j:j + 1] < top).astype(F32)
    kio = jax.lax.broadcasted_iota(jnp.int32, (B, k), 1).astype(F32)
    srt = jnp.zeros((B, k), F32)
    for i in range(k):
        srt = srt + jnp.where(rank[:, i:i + 1] == kio, top[:, i:i + 1], 0.0)
    return srt.astype(jnp.int32)


def _k3_body(ascores, pscores, ta_o, tp_o):
    ta_o[...] = _topk_select(ascores[...], KA, NA)
    tp_o[...] = _topk_select(pscores[...], KP, NP)


def _k4_body(feat, lg, acand, pcand, ascores, pscores, ta, tp, wemb,
             s1pred, s1arg, s1b, s2w, s2b, out, aspan_o, pspan_o):
    f = feat[0]
    lgr = lg[0]
    ai = ta[0, 0, :]
    pi = tp[0, 0, :]

    iota_a = jax.lax.broadcasted_iota(jnp.int32, (KA, NA), 1)
    oh_ai = (iota_a == ai[:, None]).astype(F32)
    av = jnp.concatenate([acand[0].astype(F32), ascores[0].T], axis=1)
    ag = _dot_g(oh_ai, av)
    aspan = ag[:, 0:2]
    aspan_o[0] = aspan.astype(jnp.int32)
    asc = ag[:, 2:3]

    iota_p = jax.lax.broadcasted_iota(jnp.int32, (KP, NP), 1)
    oh_pi = (iota_p == pi[:, None]).astype(F32)
    pv = jnp.concatenate([pcand[0].astype(F32), pscores[0].T], axis=1)
    pg = _dot_g(oh_pi, pv)
    pspan = pg[:, 0:2]
    pspan_o[0] = pspan.astype(jnp.int32)
    psc = pg[:, 2:3]

    s_t = aspan[:, 0:1].astype(jnp.int32)
    e_t = aspan[:, 1:2].astype(jnp.int32)
    w_t = e_t - s_t
    iota_t = jax.lax.broadcasted_iota(jnp.int32, (KA, T), 1)
    inwin = (iota_t >= s_t) & (iota_t <= e_t)
    ml = jnp.where(inwin, lgr, -1e30)
    rowmax = jnp.max(ml, axis=1, keepdims=True)
    ex = jnp.exp(ml - rowmax)
    wmat = ex / jnp.sum(ex, axis=1, keepdims=True)
    att = _dot_d(wmat, f)
    ohs = (iota_t == s_t).astype(F32)
    ohe = (iota_t == e_t).astype(F32)
    fs_t = _dot_g(ohs, f)
    fe_t = _dot_g(ohe, f)
    iw = jax.lax.broadcasted_iota(jnp.int32, (KA, NW), 1)
    ohw = (iw == w_t).astype(F32)
    wd_t = _dot_g(ohw, wemb[...])
    arg_emb = jnp.concatenate([fs_t, fe_t, wd_t, att], axis=1)
    apart = _dot_d(arg_emb, s1arg[...])

    p_t = pspan[:, 0:1].astype(jnp.int32)
    iota_tp = jax.lax.broadcasted_iota(jnp.int32, (KP, T), 1)
    ohp = (iota_tp == p_t).astype(F32)
    fp_t = _dot_g(ohp, f)
    ppart = _dot_d(jnp.concatenate([fp_t, fp_t], axis=1), s1pred[...])

    npair = KP * KA
    ip = jax.lax.broadcasted_iota(jnp.int32, (npair, KP), 0) // KA
    e1 = (ip == jax.lax.broadcasted_iota(jnp.int32, (npair, KP), 1)).astype(F32)
    ia = jax.lax.broadcasted_iota(jnp.int32, (npair, KA), 0) % KA
    e2 = (ia == jax.lax.broadcasted_iota(jnp.int32, (npair, KA), 1)).astype(F32)
    pre = _dot_g(e1, ppart) + _dot_g(e2, apart) + s1b[...]
    h = jnp.maximum(pre, 0.0)
    sc = _dot_d(h, s2w[...]) + s2b[...]
    sc = sc + _dot_g(e2, asc) + _dot_g(e1, psc)
    out[0] = jnp.concatenate([jnp.zeros((npair, 1), F32), sc], axis=1)


def _pipeline(features, arg_candidates, predicate_candidates, width_emb, att_w,
              att_b, m1w, m1b, m2w, m2b, p1w, p1b, p2w, p2b, s1w, s1b, s2w,
              s2b):
    m1ws = m1w[0:H]
    m1we = m1w[H:2 * H]
    m1ww = m1w[2 * H:2 * H + WD]
    m1wa = m1w[2 * H + WD:]
    att_b2 = att_b.reshape(1, 1)
    m1b2 = m1b.reshape(1, H)
    m2b2 = m2b.reshape(1, 1)
    p1b2 = p1b.reshape(1, H)
    p2b2 = p2b.reshape(1, 1)
    s1b2 = s1b.reshape(1, H)
    s2b2 = s2b.reshape(1, NC - 1)
    s1pred = s1w[0:2 * H]
    s1arg = s1w[2 * H:]

    full = lambda shape: pl.BlockSpec(shape, lambda b: (0,) * len(shape))
    fullt = lambda shape: pl.BlockSpec(shape, lambda b, t: (0,) * len(shape))

    boff = (jnp.arange(B, dtype=jnp.int32) * T)[:, None]
    gs_all = (arg_candidates[:, :, 0] + boff).reshape(B * NA)
    ge_all = (arg_candidates[:, :, 1] + boff).reshape(B * NA)
    feat_flat = features.reshape(B * T, H)
    ga_s, ga_e = _sc_gather(feat_flat, gs_all, ge_all)
    ga_s = ga_s.reshape(B, NA, H)
    ga_e = ga_e.reshape(B, NA, H)

    lg, psc = pl.pallas_call(
        _k1_body,
        grid=(B,),
        in_specs=[
            pl.BlockSpec((1, T, H), lambda b: (b, 0, 0)),
            full((H, 1)), full((1, 1)),
            full((2 * H, H)), full((1, H)), full((H, 1)), full((1, 1)),
            pl.BlockSpec((1, NP, 2), lambda b: (b, 0, 0)),
        ],
        out_specs=[
            pl.BlockSpec((1, 1, T), lambda b: (b, 0, 0)),
            pl.BlockSpec((1, 1, NP), lambda b: (b, 0, 0)),
        ],
        out_shape=[
            jax.ShapeDtypeStruct((B, 1, T), F32),
            jax.ShapeDtypeStruct((B, 1, NP), F32),
        ],
    )(features, att_w, att_b2, p1w, p1b2, p2w, p2b2, predicate_candidates)

    ascores = pl.pallas_call(
        _k2_body,
        grid=(B, NA // NT),
        in_specs=[
            pl.BlockSpec((1, T, H), lambda b, t: (b, 0, 0)),
            pl.BlockSpec((1, NT, H), lambda b, t: (b, t, 0)),
            pl.BlockSpec((1, NT, H), lambda b, t: (b, t, 0)),
            fullt((NW, WD)), fullt((H, H)), fullt((H, H)),
            fullt((WD, H)), fullt((H, H)),
            pl.BlockSpec((1, 1, T), lambda b, t: (b, 0, 0)),
            pl.BlockSpec((1, NT, 2), lambda b, t: (b, t, 0)),
            fullt((1, H)), fullt((H, 1)), fullt((1, 1)),
        ],
        out_specs=pl.BlockSpec((1, 1, NT), lambda b, t: (b, 0, t)),
        out_shape=jax.ShapeDtypeStruct((B, 1, NA), F32),
    )(features, ga_s, ga_e, width_emb, m1ws, m1we, m1ww, m1wa, lg,
      arg_candidates, m1b2, m2w, m2b2)

    ta, tp = pl.pallas_call(
        _k3_body,
        in_specs=[pl.BlockSpec((B, NA), lambda: (0, 0)),
                  pl.BlockSpec((B, NP), lambda: (0, 0))],
        out_specs=[pl.BlockSpec((B, KA), lambda: (0, 0)),
                   pl.BlockSpec((B, KP), lambda: (0, 0))],
        out_shape=[jax.ShapeDtypeStruct((B, KA), jnp.int32),
                   jax.ShapeDtypeStruct((B, KP), jnp.int32)],
    )(ascores.reshape(B, NA), psc.reshape(B, NP))

    out, aspan, pspan = pl.pallas_call(
        _k4_body,
        grid=(B,),
        in_specs=[
            pl.BlockSpec((1, T, H), lambda b: (b, 0, 0)),
            pl.BlockSpec((1, 1, T), lambda b: (b, 0, 0)),
            pl.BlockSpec((1, NA, 2), lambda b: (b, 0, 0)),
            pl.BlockSpec((1, NP, 2), lambda b: (b, 0, 0)),
            pl.BlockSpec((1, 1, NA), lambda b: (b, 0, 0)),
            pl.BlockSpec((1, 1, NP), lambda b: (b, 0, 0)),
            pl.BlockSpec((1, 1, KA), lambda b: (b, 0, 0)),
            pl.BlockSpec((1, 1, KP), lambda b: (b, 0, 0)),
            full((NW, WD)),
            full((2 * H, H)), full((3 * H + WD, H)),
            full((1, H)), full((H, NC - 1)), full((1, NC - 1)),
        ],
        out_specs=[
            pl.BlockSpec((1, KP * KA, NC), lambda b: (b, 0, 0)),
            pl.BlockSpec((1, KA, 2), lambda b: (b, 0, 0)),
            pl.BlockSpec((1, KP, 2), lambda b: (b, 0, 0)),
        ],
        out_shape=[
            jax.ShapeDtypeStruct((B, KP * KA, NC), F32),
            jax.ShapeDtypeStruct((B, KA, 2), jnp.int32),
            jax.ShapeDtypeStruct((B, KP, 2), jnp.int32),
        ],
    )(features, lg, arg_candidates, predicate_candidates, ascores,
      psc, ta.reshape(B, 1, KA), tp.reshape(B, 1, KP), width_emb,
      s1pred, s1arg, s1b2, s2w, s2b2)

    return (out.reshape(B, KP, KA, NC), pspan, aspan), (ascores, psc, ta, tp)


def kernel(features, arg_candidates, predicate_candidates, width_emb, att_w,
           att_b, m1w, m1b, m2w, m2b, p1w, p1b, p2w, p2b, s1w, s1b, s2w, s2b):
    outs, _ = _pipeline(features, arg_candidates, predicate_candidates,
                        width_emb, att_w, att_b, m1w, m1b, m2w, m2b, p1w, p1b,
                        p2w, p2b, s1w, s1b, s2w, s2b)
    return outs
